# baseline (speedup 1.0000x reference)
import functools

import jax
import jax.numpy as jnp
from jax import lax
from jax.experimental import pallas as pl
from jax.experimental.pallas import tpu as pltpu
from jax.experimental.pallas import tpu_sc as plsc

N = 10000
E = 160000
H = 128

NC = 2
NS = 16
NW = NC * NS
EPW = E // NW
K = 125
NB = EPW // K
NPAD = 10240
RPT = NPAD // NS
DPT = NPAD // NS


NBD = E // NS // K


def _deg_body(dst_hbm, ones_hbm, zeros_hbm, out_hbm, idx_v, ones_v, acc_sh, sem):
    c = lax.axis_index("c")
    s = lax.axis_index("s")

    @pl.when(c == 0)
    def _():
        pltpu.sync_copy(dst_hbm.at[s], idx_v)
        pltpu.sync_copy(ones_hbm, ones_v)
        pltpu.sync_copy(zeros_hbm, acc_sh.at[pl.ds(s * DPT, DPT)])
        plsc.subcore_barrier()

        def body(j, carry):
            pltpu.sync_copy(ones_v, acc_sh.at[idx_v.at[j]], add=True)
            return carry

        lax.fori_loop(0, NBD, body, 0)
        plsc.subcore_barrier()
        pltpu.sync_copy(acc_sh.at[pl.ds(s * DPT, DPT)],
                        out_hbm.at[pl.ds(s * DPT, DPT)])


_deg_kernel = functools.partial(
    pl.kernel,
    out_type=jax.ShapeDtypeStruct((NPAD,), jnp.float32),
    mesh=plsc.VectorSubcoreMesh(core_axis_name="c", subcore_axis_name="s", num_cores=NC, num_subcores=NS),
    scratch_types=[
        pltpu.VMEM((NBD, K), jnp.int32),
        pltpu.VMEM((K,), jnp.float32),
        pltpu.VMEM_SHARED((NPAD,), jnp.float32),
        pltpu.SemaphoreType.DMA,
    ],
)(_deg_body)


def _agg_body(zp_hbm, src_hbm, dst_hbm, zrows_hbm, out_hbm,
              sidx, didx, rows0, rows1, acc_sh, gsem0, gsem1, ssem0, ssem1):
    c = lax.axis_index("c")
    s = lax.axis_index("s")
    w = c * NS + s
    pltpu.sync_copy(src_hbm.at[w], sidx)
    pltpu.sync_copy(dst_hbm.at[w], didx)
    pltpu.sync_copy(zrows_hbm, acc_sh.at[pl.ds(s * RPT, RPT)])
    plsc.subcore_barrier()

    pltpu.async_copy(zp_hbm.at[sidx.at[0]], rows0, gsem0)
    pltpu.async_copy(zp_hbm.at[sidx.at[1]], rows1, gsem1)

    def body(t, carry):
        j0 = 2 * t
        pltpu.make_async_copy(zp_hbm.at[sidx.at[j0]], rows0, gsem0).wait()
        pltpu.async_copy(rows0, acc_sh.at[didx.at[j0]], ssem0, add=True)
        pltpu.make_async_copy(zp_hbm.at[sidx.at[j0 + 1]], rows1, gsem1).wait()
        pltpu.async_copy(rows1, acc_sh.at[didx.at[j0 + 1]], ssem1, add=True)

        @pl.when(t < NB // 2 - 1)
        def _():
            pltpu.make_async_copy(
                rows0, acc_sh.at[didx.at[j0]], ssem0).wait()
            pltpu.async_copy(zp_hbm.at[sidx.at[j0 + 2]], rows0, gsem0)
            pltpu.make_async_copy(
                rows1, acc_sh.at[didx.at[j0 + 1]], ssem1).wait()
            pltpu.async_copy(zp_hbm.at[sidx.at[j0 + 3]], rows1, gsem1)

        return carry

    lax.fori_loop(0, NB // 2, body, 0)
    pltpu.make_async_copy(rows0, acc_sh.at[didx.at[NB - 2]], ssem0).wait()
    pltpu.make_async_copy(rows1, acc_sh.at[didx.at[NB - 1]], ssem1).wait()
    plsc.subcore_barrier()
    base = c * NPAD + s * RPT
    pltpu.sync_copy(acc_sh.at[pl.ds(s * RPT, RPT)], out_hbm.at[pl.ds(base, RPT)])


_agg_kernel = functools.partial(
    pl.kernel,
    out_type=jax.ShapeDtypeStruct((NC * NPAD, H), jnp.float32),
    mesh=plsc.VectorSubcoreMesh(core_axis_name="c", subcore_axis_name="s", num_cores=NC, num_subcores=NS),
    scratch_types=[
        pltpu.VMEM((NB, K), jnp.int32),
        pltpu.VMEM((NB, K), jnp.int32),
        pltpu.VMEM((K, H), jnp.float32),
        pltpu.VMEM((K, H), jnp.float32),
        pltpu.VMEM_SHARED((NPAD, H), jnp.float32),
        pltpu.SemaphoreType.DMA,
        pltpu.SemaphoreType.DMA,
        pltpu.SemaphoreType.DMA,
        pltpu.SemaphoreType.DMA,
    ],
)(_agg_body)


BR = 1000


def _fast_sin(x):
    k = jnp.round(x * (1.0 / jnp.pi))
    r = x - k * jnp.pi
    ki = k.astype(jnp.int32)
    sgn = jnp.where((ki & 1) == 0, 1.0, -1.0)
    r2 = r * r
    p = r * (1.0 + r2 * (-0.16666667 + r2 * (8.3333310e-3
                                             + r2 * (-1.9840874e-4
                                                     + r2 * 2.7525562e-6))))
    return sgn * p


def _lin1_body(x_ref, xyz_ref, rot_ref, col_ref, sc_ref, op_ref,
               wx_ref, wf_ref, scales_ref, degp_ref, zp_ref, dinv_ref):
    deg = degp_ref[...] + 1.0
    dinv = lax.rsqrt(deg)
    pos = _fast_sin(xyz_ref[...] * scales_ref[...])
    feat = jnp.concatenate(
        [pos, rot_ref[...], col_ref[...], sc_ref[...], op_ref[...]], axis=1)
    z = (jnp.dot(x_ref[...], wx_ref[...], preferred_element_type=jnp.float32)
         + jnp.dot(feat.astype(jnp.bfloat16), wf_ref[...],
                   preferred_element_type=jnp.float32))
    zp_ref[...] = z * dinv
    dinv_ref[...] = dinv


def _lin1(x, xyz, rot, color, scale, opacity, w1x, w1f, scales, degp1):
    row = lambda i: (i, 0)
    return pl.pallas_call(
        _lin1_body,
        grid=(N // BR,),
        in_specs=[
            pl.BlockSpec((BR, 288), row),
            pl.BlockSpec((BR, 3), row),
            pl.BlockSpec((BR, 4), row),
            pl.BlockSpec((BR, 3), row),
            pl.BlockSpec((BR, 3), row),
            pl.BlockSpec((BR, 1), row),
            pl.BlockSpec((288, H), lambda i: (0, 0)),
            pl.BlockSpec((14, H), lambda i: (0, 0)),
            pl.BlockSpec((1, 3), lambda i: (0, 0)),
            pl.BlockSpec((BR, 1), row),
        ],
        out_specs=[
            pl.BlockSpec((BR, H), row),
            pl.BlockSpec((BR, 1), row),
        ],
        out_shape=[
            jax.ShapeDtypeStruct((N, H), jnp.float32),
            jax.ShapeDtypeStruct((N, 1), jnp.float32),
        ],
    )(x, xyz, rot, color, scale, opacity, w1x, w1f, scales, degp1)


def _mid_body(acc_ref, zp_ref, dinv_ref, b_ref, w_ref, out_ref):
    dinv = dinv_ref[...]
    hcur = jnp.maximum(
        dinv * (acc_ref[0] + acc_ref[1] + zp_ref[...]) + b_ref[...], 0.0)
    z = jnp.dot(hcur, w_ref[...], preferred_element_type=jnp.float32)
    out_ref[...] = z * dinv


def _mid(acc, zp_prev, dinv, b, w):
    return pl.pallas_call(
        _mid_body,
        grid=(N // BR,),
        in_specs=[
            pl.BlockSpec((2, BR, H), lambda i: (0, i, 0)),
            pl.BlockSpec((BR, H), lambda i: (i, 0)),
            pl.BlockSpec((BR, 1), lambda i: (i, 0)),
            pl.BlockSpec((1, H), lambda i: (0, 0)),
            pl.BlockSpec((H, H), lambda i: (0, 0)),
        ],
        out_specs=pl.BlockSpec((BR, H), lambda i: (i, 0)),
        out_shape=jax.ShapeDtypeStruct((N, H), jnp.float32),
    )(acc, zp_prev, dinv, b, w)


def _heads_body(acc_ref, zp_ref, dinv_ref, b3_ref, w1c_ref, b1c_ref,
                w2b_ref, b2c_ref, st_ref, dscale_ref, lo_ref, hi_ref,
                xyz_o, rot_o, col_o, sc_o, op_o):
    dinv = dinv_ref[...]
    h3 = jnp.maximum(
        dinv * (acc_ref[0] + acc_ref[1] + zp_ref[...]) + b3_ref[...], 0.0)
    hh = jnp.maximum(
        jnp.dot(h3.astype(jnp.bfloat16), w1c_ref[...],
                preferred_element_type=jnp.float32)
        + b1c_ref[...], 0.0)
    delta = (jnp.dot(hh.astype(jnp.bfloat16), w2b_ref[...],
                     preferred_element_type=jnp.float32)
             + b2c_ref[...])
    new = st_ref[...] + delta * dscale_ref[...]
    out = jnp.clip(new, lo_ref[...], hi_ref[...])
    rn = new[:, 3:7]
    msk = (lax.broadcasted_iota(jnp.int32, new.shape, 1) - 3).astype(jnp.uint32) < 4
    nrm = jnp.sqrt(jnp.sum(jnp.where(msk, new * new, 0.0), axis=1,
                           keepdims=True))
    xyz_o[...] = out[:, 0:3]
    rot_o[...] = rn / jnp.clip(nrm, 1e-12, None)
    col_o[...] = out[:, 7:10]
    sc_o[...] = out[:, 10:13]
    op_o[...] = out[:, 13:14]


def _heads(acc, zp_prev, dinv, b3, w1c, b1c, w2b, b2c, state, dscale, lo, hi):
    row = lambda i: (i, 0)
    full = lambda shp: pl.BlockSpec(shp, lambda i: (0, 0))
    return pl.pallas_call(
        _heads_body,
        grid=(N // BR,),
        in_specs=[
            pl.BlockSpec((2, BR, H), lambda i: (0, i, 0)),
            pl.BlockSpec((BR, H), row),
            pl.BlockSpec((BR, 1), row),
            full((1, H)),
            full((H, 320)),
            full((1, 320)),
            full((320, 14)),
            full((1, 14)),
            pl.BlockSpec((BR, 14), row),
            full((1, 14)),
            full((1, 14)),
            full((1, 14)),
        ],
        out_specs=[
            pl.BlockSpec((BR, 3), row),
            pl.BlockSpec((BR, 4), row),
            pl.BlockSpec((BR, 3), row),
            pl.BlockSpec((BR, 3), row),
            pl.BlockSpec((BR, 1), row),
        ],
        out_shape=[
            jax.ShapeDtypeStruct((N, 3), jnp.float32),
            jax.ShapeDtypeStruct((N, 4), jnp.float32),
            jax.ShapeDtypeStruct((N, 3), jnp.float32),
            jax.ShapeDtypeStruct((N, 3), jnp.float32),
            jax.ShapeDtypeStruct((N, 1), jnp.float32),
        ],
    )(acc, zp_prev, dinv, b3, w1c, b1c, w2b, b2c, state, dscale, lo, hi)


def kernel(x, xyz, rot, color, scale, opacity, edge_index,
           W1, b1, W2, b2, W3, b3,
           Wp1, bp1, Wp2, bp2, Wr1, br1, Wr2, br2,
           Wc1, bc1, Wc2, bc2, Ws1, bs1, Ws2, bs2,
           Wo1, bo1, Wo2, bo2):
    src_r = edge_index[0].reshape(NW, NB, K)
    dst_r = edge_index[1].reshape(NW, NB, K)
    dst_d = edge_index[1].reshape(NS, NBD, K)
    ones_k = jnp.ones((K,), jnp.float32)
    zeros_d = jnp.zeros((DPT,), jnp.float32)
    zeros_rows = jnp.zeros((RPT, H), jnp.float32)
    x_bf = x.astype(jnp.bfloat16)
    w1x = W1[:288].astype(jnp.bfloat16)
    w1f = W1[288:].astype(jnp.bfloat16)

    w1c = jnp.concatenate(
        [Wp1, Wr1, Wc1, Ws1, Wo1], axis=1).astype(jnp.bfloat16)
    b1c = jnp.concatenate([bp1, br1, bc1, bs1, bo1])[None, :]
    w2b = jnp.zeros((320, 14), jnp.float32)
    w2b = w2b.at[0:64, 0:3].set(Wp2)
    w2b = w2b.at[64:128, 3:7].set(Wr2)
    w2b = w2b.at[128:192, 7:10].set(Wc2)
    w2b = w2b.at[192:256, 10:13].set(Ws2)
    w2b = w2b.at[256:320, 13:14].set(Wo2)
    w2b = w2b.astype(jnp.bfloat16)
    b2c = jnp.concatenate([bp2, br2, bc2, bs2, bo2])[None, :]

    degp1 = _deg_kernel(dst_d, ones_k, zeros_d)[:, None]

    scales = jnp.array([[10.0, 20.0, 30.0]], jnp.float32)
    z1p, dinv = _lin1(x_bf, xyz, rot, color, scale, opacity, w1x, w1f,
                      scales, degp1)
    acc1 = _agg_kernel(z1p, src_r, dst_r, zeros_rows).reshape(NC, NPAD, H)
    z2p = _mid(acc1, z1p, dinv, b1[None, :], W2)
    acc2 = _agg_kernel(z2p, src_r, dst_r, zeros_rows).reshape(NC, NPAD, H)
    z3p = _mid(acc2, z2p, dinv, b2[None, :], W3)
    acc3 = _agg_kernel(z3p, src_r, dst_r, zeros_rows).reshape(NC, NPAD, H)
    state = jnp.concatenate([xyz, rot, color, scale, opacity], axis=1)
    inf = jnp.inf
    dscale = jnp.array([[0.1] * 3 + [1.0] * 4 + [1.0] * 3 + [0.1] * 3 + [1.0]],
                       jnp.float32)
    lo = jnp.array([[-inf] * 7 + [0.0] * 3 + [1e-6] * 3 + [0.0]], jnp.float32)
    hi = jnp.array([[inf] * 7 + [1.0] * 3 + [1000.0] * 3 + [1.0]], jnp.float32)
    return _heads(acc3, z3p, dinv, b3[None, :], w1c, b1c, w2b, b2c,
                  state, dscale, lo, hi)


# speedup vs baseline: 1.0725x; 1.0725x over previous
import functools

import jax
import jax.numpy as jnp
from jax import lax
from jax.experimental import pallas as pl
from jax.experimental.pallas import tpu as pltpu
from jax.experimental.pallas import tpu_sc as plsc

N = 10000
E = 160000
H = 128

NC = 2
NS = 16
NW = NC * NS
EPW = E // NW
K = 125
NB = EPW // K
NPAD = 10240
RPT = NPAD // NS
DPT = NPAD // NS


NBD = E // NS // K


def _deg_body(dst_hbm, ones_hbm, zeros_hbm, out_hbm, idx_v, ones_v, acc_sh, sem):
    c = lax.axis_index("c")
    s = lax.axis_index("s")

    @pl.when(c == 0)
    def _():
        pltpu.sync_copy(dst_hbm.at[s], idx_v)
        pltpu.sync_copy(ones_hbm, ones_v)
        pltpu.sync_copy(zeros_hbm, acc_sh.at[pl.ds(s * DPT, DPT)])
        plsc.subcore_barrier()

        def body(j, carry):
            pltpu.sync_copy(ones_v, acc_sh.at[idx_v.at[j]], add=True)
            return carry

        lax.fori_loop(0, NBD, body, 0)
        plsc.subcore_barrier()
        pltpu.sync_copy(acc_sh.at[pl.ds(s * DPT, DPT)],
                        out_hbm.at[pl.ds(s * DPT, DPT)])


_deg_kernel = functools.partial(
    pl.kernel,
    out_type=jax.ShapeDtypeStruct((NPAD,), jnp.float32),
    mesh=plsc.VectorSubcoreMesh(core_axis_name="c", subcore_axis_name="s", num_cores=NC, num_subcores=NS),
    scratch_types=[
        pltpu.VMEM((NBD, K), jnp.int32),
        pltpu.VMEM((K,), jnp.float32),
        pltpu.VMEM_SHARED((NPAD,), jnp.float32),
        pltpu.SemaphoreType.DMA,
    ],
)(_deg_body)


def _agg_body(zp_hbm, src_hbm, dst_hbm, zrows_hbm, out_hbm,
              sidx, didx, rows0, rows1, acc_sh, gsem0, gsem1):
    c = lax.axis_index("c")
    s = lax.axis_index("s")
    w = c * NS + s
    pltpu.sync_copy(src_hbm.at[w], sidx)
    pltpu.sync_copy(dst_hbm.at[w], didx)

    @pl.when(c == 0)
    def _():
        pltpu.sync_copy(zp_hbm.at[pl.ds(s * RPT, RPT)],
                        acc_sh.at[pl.ds(s * RPT, RPT)])

    @pl.when(c == 1)
    def _():
        pltpu.sync_copy(zrows_hbm, acc_sh.at[pl.ds(s * RPT, RPT)])

    plsc.subcore_barrier()

    pltpu.async_copy(zp_hbm.at[sidx.at[0]], rows0, gsem0)

    def body(t, carry):
        j0 = 2 * t
        pltpu.make_async_copy(zp_hbm.at[sidx.at[j0]], rows0, gsem0).wait()
        pltpu.async_copy(zp_hbm.at[sidx.at[j0 + 1]], rows1, gsem1)
        pltpu.sync_copy(rows0, acc_sh.at[didx.at[j0]], add=True)
        pltpu.make_async_copy(zp_hbm.at[sidx.at[j0 + 1]], rows1, gsem1).wait()

        @pl.when(t < NB // 2 - 1)
        def _():
            pltpu.async_copy(zp_hbm.at[sidx.at[j0 + 2]], rows0, gsem0)

        pltpu.sync_copy(rows1, acc_sh.at[didx.at[j0 + 1]], add=True)
        return carry

    lax.fori_loop(0, NB // 2, body, 0)
    plsc.subcore_barrier()
    base = c * NPAD + s * RPT
    pltpu.sync_copy(acc_sh.at[pl.ds(s * RPT, RPT)], out_hbm.at[pl.ds(base, RPT)])


_agg_kernel = functools.partial(
    pl.kernel,
    out_type=jax.ShapeDtypeStruct((NC * NPAD, H), jnp.float32),
    mesh=plsc.VectorSubcoreMesh(core_axis_name="c", subcore_axis_name="s", num_cores=NC, num_subcores=NS),
    scratch_types=[
        pltpu.VMEM((NB, K), jnp.int32),
        pltpu.VMEM((NB, K), jnp.int32),
        pltpu.VMEM((K, H), jnp.float32),
        pltpu.VMEM((K, H), jnp.float32),
        pltpu.VMEM_SHARED((NPAD, H), jnp.float32),
        pltpu.SemaphoreType.DMA,
        pltpu.SemaphoreType.DMA,
    ],
)(_agg_body)


BR = 1000


def _fast_sin(x):
    k = jnp.round(x * (1.0 / jnp.pi))
    r = x - k * jnp.pi
    ki = k.astype(jnp.int32)
    sgn = jnp.where((ki & 1) == 0, 1.0, -1.0)
    r2 = r * r
    p = r * (1.0 + r2 * (-0.16666667 + r2 * (8.3333310e-3
                                             + r2 * (-1.9840874e-4
                                                     + r2 * 2.7525562e-6))))
    return sgn * p


def _lin1_body(x_ref, xyz_ref, rot_ref, col_ref, sc_ref, op_ref,
               wx_ref, wf_ref, scales_ref, degp_ref, zp_ref, dinv_ref):
    deg = degp_ref[...] + 1.0
    dinv = lax.rsqrt(deg)
    pos = _fast_sin(xyz_ref[...] * scales_ref[...])
    feat = jnp.concatenate(
        [pos, rot_ref[...], col_ref[...], sc_ref[...], op_ref[...]], axis=1)
    z = (jnp.dot(x_ref[...], wx_ref[...], preferred_element_type=jnp.float32)
         + jnp.dot(feat.astype(jnp.bfloat16), wf_ref[...],
                   preferred_element_type=jnp.float32))
    zp_ref[...] = z * dinv
    dinv_ref[...] = dinv


def _lin1(x, xyz, rot, color, scale, opacity, w1x, w1f, scales, degp1):
    row = lambda i: (i, 0)
    return pl.pallas_call(
        _lin1_body,
        grid=(N // BR,),
        in_specs=[
            pl.BlockSpec((BR, 288), row),
            pl.BlockSpec((BR, 3), row),
            pl.BlockSpec((BR, 4), row),
            pl.BlockSpec((BR, 3), row),
            pl.BlockSpec((BR, 3), row),
            pl.BlockSpec((BR, 1), row),
            pl.BlockSpec((288, H), lambda i: (0, 0)),
            pl.BlockSpec((14, H), lambda i: (0, 0)),
            pl.BlockSpec((1, 3), lambda i: (0, 0)),
            pl.BlockSpec((BR, 1), row),
        ],
        out_specs=[
            pl.BlockSpec((BR, H), row),
            pl.BlockSpec((BR, 1), row),
        ],
        out_shape=[
            jax.ShapeDtypeStruct((NPAD, H), jnp.float32),
            jax.ShapeDtypeStruct((NPAD, 1), jnp.float32),
        ],
    )(x, xyz, rot, color, scale, opacity, w1x, w1f, scales, degp1)


def _mid_body(acc_ref, dinv_ref, b_ref, w_ref, out_ref):
    dinv = dinv_ref[...]
    hcur = jnp.maximum(
        dinv * (acc_ref[0] + acc_ref[1]) + b_ref[...], 0.0)
    z = jnp.dot(hcur, w_ref[...], preferred_element_type=jnp.float32)
    out_ref[...] = z * dinv


def _mid(acc, dinv, b, w):
    return pl.pallas_call(
        _mid_body,
        grid=(N // BR,),
        in_specs=[
            pl.BlockSpec((2, BR, H), lambda i: (0, i, 0)),
            pl.BlockSpec((BR, 1), lambda i: (i, 0)),
            pl.BlockSpec((1, H), lambda i: (0, 0)),
            pl.BlockSpec((H, H), lambda i: (0, 0)),
        ],
        out_specs=pl.BlockSpec((BR, H), lambda i: (i, 0)),
        out_shape=jax.ShapeDtypeStruct((NPAD, H), jnp.float32),
    )(acc, dinv, b, w)


def _heads_body(acc_ref, dinv_ref, b3_ref, w1c_ref, b1c_ref,
                w2b_ref, b2c_ref, st_ref, dscale_ref, lo_ref, hi_ref,
                xyz_o, rot_o, col_o, sc_o, op_o):
    dinv = dinv_ref[...]
    h3 = jnp.maximum(
        dinv * (acc_ref[0] + acc_ref[1]) + b3_ref[...], 0.0)
    hh = jnp.maximum(
        jnp.dot(h3.astype(jnp.bfloat16), w1c_ref[...],
                preferred_element_type=jnp.float32)
        + b1c_ref[...], 0.0)
    delta = (jnp.dot(hh.astype(jnp.bfloat16), w2b_ref[...],
                     preferred_element_type=jnp.float32)
             + b2c_ref[...])
    new = st_ref[...] + delta * dscale_ref[...]
    out = jnp.clip(new, lo_ref[...], hi_ref[...])
    rn = new[:, 3:7]
    msk = (lax.broadcasted_iota(jnp.int32, new.shape, 1) - 3).astype(jnp.uint32) < 4
    nrm = jnp.sqrt(jnp.sum(jnp.where(msk, new * new, 0.0), axis=1,
                           keepdims=True))
    xyz_o[...] = out[:, 0:3]
    rot_o[...] = rn / jnp.clip(nrm, 1e-12, None)
    col_o[...] = out[:, 7:10]
    sc_o[...] = out[:, 10:13]
    op_o[...] = out[:, 13:14]


def _heads(acc, dinv, b3, w1c, b1c, w2b, b2c, state, dscale, lo, hi):
    row = lambda i: (i, 0)
    full = lambda shp: pl.BlockSpec(shp, lambda i: (0, 0))
    return pl.pallas_call(
        _heads_body,
        grid=(N // BR,),
        in_specs=[
            pl.BlockSpec((2, BR, H), lambda i: (0, i, 0)),
            pl.BlockSpec((BR, 1), row),
            full((1, H)),
            full((H, 320)),
            full((1, 320)),
            full((320, 14)),
            full((1, 14)),
            pl.BlockSpec((BR, 14), row),
            full((1, 14)),
            full((1, 14)),
            full((1, 14)),
        ],
        out_specs=[
            pl.BlockSpec((BR, 3), row),
            pl.BlockSpec((BR, 4), row),
            pl.BlockSpec((BR, 3), row),
            pl.BlockSpec((BR, 3), row),
            pl.BlockSpec((BR, 1), row),
        ],
        out_shape=[
            jax.ShapeDtypeStruct((N, 3), jnp.float32),
            jax.ShapeDtypeStruct((N, 4), jnp.float32),
            jax.ShapeDtypeStruct((N, 3), jnp.float32),
            jax.ShapeDtypeStruct((N, 3), jnp.float32),
            jax.ShapeDtypeStruct((N, 1), jnp.float32),
        ],
    )(acc, dinv, b3, w1c, b1c, w2b, b2c, state, dscale, lo, hi)


def kernel(x, xyz, rot, color, scale, opacity, edge_index,
           W1, b1, W2, b2, W3, b3,
           Wp1, bp1, Wp2, bp2, Wr1, br1, Wr2, br2,
           Wc1, bc1, Wc2, bc2, Ws1, bs1, Ws2, bs2,
           Wo1, bo1, Wo2, bo2):
    src_r = edge_index[0].reshape(NW, NB, K)
    dst_r = edge_index[1].reshape(NW, NB, K)
    dst_d = edge_index[1].reshape(NS, NBD, K)
    ones_k = jnp.ones((K,), jnp.float32)
    zeros_d = jnp.zeros((DPT,), jnp.float32)
    zeros_rows = jnp.zeros((RPT, H), jnp.float32)
    x_bf = x.astype(jnp.bfloat16)
    w1x = W1[:288].astype(jnp.bfloat16)
    w1f = W1[288:].astype(jnp.bfloat16)

    w1c = jnp.concatenate(
        [Wp1, Wr1, Wc1, Ws1, Wo1], axis=1).astype(jnp.bfloat16)
    b1c = jnp.concatenate([bp1, br1, bc1, bs1, bo1])[None, :]
    w2b = jnp.zeros((320, 14), jnp.float32)
    w2b = w2b.at[0:64, 0:3].set(Wp2)
    w2b = w2b.at[64:128, 3:7].set(Wr2)
    w2b = w2b.at[128:192, 7:10].set(Wc2)
    w2b = w2b.at[192:256, 10:13].set(Ws2)
    w2b = w2b.at[256:320, 13:14].set(Wo2)
    w2b = w2b.astype(jnp.bfloat16)
    b2c = jnp.concatenate([bp2, br2, bc2, bs2, bo2])[None, :]

    degp1 = _deg_kernel(dst_d, ones_k, zeros_d)[:, None]

    scales = jnp.array([[10.0, 20.0, 30.0]], jnp.float32)
    z1p, dinv = _lin1(x_bf, xyz, rot, color, scale, opacity, w1x, w1f,
                      scales, degp1)
    acc1 = _agg_kernel(z1p, src_r, dst_r, zeros_rows).reshape(NC, NPAD, H)
    z2p = _mid(acc1, dinv, b1[None, :], W2)
    acc2 = _agg_kernel(z2p, src_r, dst_r, zeros_rows).reshape(NC, NPAD, H)
    z3p = _mid(acc2, dinv, b2[None, :], W3)
    acc3 = _agg_kernel(z3p, src_r, dst_r, zeros_rows).reshape(NC, NPAD, H)
    state = jnp.concatenate([xyz, rot, color, scale, opacity], axis=1)
    inf = jnp.inf
    dscale = jnp.array([[0.1] * 3 + [1.0] * 4 + [1.0] * 3 + [0.1] * 3 + [1.0]],
                       jnp.float32)
    lo = jnp.array([[-inf] * 7 + [0.0] * 3 + [1e-6] * 3 + [0.0]], jnp.float32)
    hi = jnp.array([[inf] * 7 + [1.0] * 3 + [1000.0] * 3 + [1.0]], jnp.float32)
    return _heads(acc3, dinv, b3[None, :], w1c, b1c, w2b, b2c,
                  state, dscale, lo, hi)


# speedup vs baseline: 1.0731x; 1.0006x over previous
import functools

import jax
import jax.numpy as jnp
from jax import lax
from jax.experimental import pallas as pl
from jax.experimental.pallas import tpu as pltpu
from jax.experimental.pallas import tpu_sc as plsc

N = 10000
E = 160000
H = 128

NC = 2
NS = 16
NW = NC * NS
EPW = E // NW
K = 125
NB = EPW // K
NPAD = 10240
RPT = NPAD // NS
DPT = NPAD // NS


NBD = E // NS // K


def _deg_body(dst_hbm, ones_hbm, zeros_hbm, out_hbm, idx_v, ones_v, acc_sh, sem):
    c = lax.axis_index("c")
    s = lax.axis_index("s")

    @pl.when(c == 0)
    def _():
        pltpu.sync_copy(dst_hbm.at[s], idx_v)
        pltpu.sync_copy(ones_hbm, ones_v)
        pltpu.sync_copy(zeros_hbm, acc_sh.at[pl.ds(s * DPT, DPT)])
        plsc.subcore_barrier()

        def body(j, carry):
            pltpu.sync_copy(ones_v, acc_sh.at[idx_v.at[j]], add=True)
            return carry

        lax.fori_loop(0, NBD, body, 0)
        plsc.subcore_barrier()
        pltpu.sync_copy(acc_sh.at[pl.ds(s * DPT, DPT)],
                        out_hbm.at[pl.ds(s * DPT, DPT)])


_deg_kernel = functools.partial(
    pl.kernel,
    out_type=jax.ShapeDtypeStruct((NPAD,), jnp.float32),
    mesh=plsc.VectorSubcoreMesh(core_axis_name="c", subcore_axis_name="s", num_cores=NC, num_subcores=NS),
    scratch_types=[
        pltpu.VMEM((NBD, K), jnp.int32),
        pltpu.VMEM((K,), jnp.float32),
        pltpu.VMEM_SHARED((NPAD,), jnp.float32),
        pltpu.SemaphoreType.DMA,
    ],
)(_deg_body)


def _agg_body(zp_hbm, src_hbm, dst_hbm, zrows_hbm, out_hbm,
              sidx, didx, rows0, rows1, acc_sh, gsem0, gsem1):
    c = lax.axis_index("c")
    s = lax.axis_index("s")
    w = c * NS + s
    pltpu.sync_copy(src_hbm.at[w], sidx)
    pltpu.sync_copy(dst_hbm.at[w], didx)

    @pl.when(c == 0)
    def _():
        pltpu.sync_copy(zp_hbm.at[pl.ds(s * RPT, RPT)],
                        acc_sh.at[pl.ds(s * RPT, RPT)])

    @pl.when(c == 1)
    def _():
        pltpu.sync_copy(zrows_hbm, acc_sh.at[pl.ds(s * RPT, RPT)])

    plsc.subcore_barrier()

    pltpu.async_copy(zp_hbm.at[sidx.at[0]], rows0, gsem0)

    def body(t, carry):
        j0 = 2 * t
        pltpu.make_async_copy(zp_hbm.at[sidx.at[j0]], rows0, gsem0).wait()
        pltpu.async_copy(zp_hbm.at[sidx.at[j0 + 1]], rows1, gsem1)
        pltpu.sync_copy(rows0, acc_sh.at[didx.at[j0]], add=True)
        pltpu.make_async_copy(zp_hbm.at[sidx.at[j0 + 1]], rows1, gsem1).wait()

        @pl.when(t < NB // 2 - 1)
        def _():
            pltpu.async_copy(zp_hbm.at[sidx.at[j0 + 2]], rows0, gsem0)

        pltpu.sync_copy(rows1, acc_sh.at[didx.at[j0 + 1]], add=True)
        return carry

    lax.fori_loop(0, NB // 2, body, 0)
    plsc.subcore_barrier()
    base = c * NPAD + s * RPT
    pltpu.sync_copy(acc_sh.at[pl.ds(s * RPT, RPT)], out_hbm.at[pl.ds(base, RPT)])


_agg_kernel = functools.partial(
    pl.kernel,
    out_type=jax.ShapeDtypeStruct((NC * NPAD, H), jnp.float32),
    mesh=plsc.VectorSubcoreMesh(core_axis_name="c", subcore_axis_name="s", num_cores=NC, num_subcores=NS),
    scratch_types=[
        pltpu.VMEM((NB, K), jnp.int32),
        pltpu.VMEM((NB, K), jnp.int32),
        pltpu.VMEM((K, H), jnp.float32),
        pltpu.VMEM((K, H), jnp.float32),
        pltpu.VMEM_SHARED((NPAD, H), jnp.float32),
        pltpu.SemaphoreType.DMA,
        pltpu.SemaphoreType.DMA,
    ],
)(_agg_body)


BR = 1000


def _fast_sin(x):
    k = jnp.round(x * (1.0 / jnp.pi))
    r = x - k * jnp.pi
    ki = k.astype(jnp.int32)
    sgn = jnp.where((ki & 1) == 0, 1.0, -1.0)
    r2 = r * r
    p = r * (1.0 + r2 * (-0.16666667 + r2 * (8.3333310e-3
                                             + r2 * (-1.9840874e-4
                                                     + r2 * 2.7525562e-6))))
    return sgn * p


def _lin1_body(x_ref, xyz_ref, rot_ref, col_ref, sc_ref, op_ref,
               wx_ref, wf_ref, scales_ref, degp_ref, zp_ref, dinv_ref):
    deg = degp_ref[...] + 1.0
    dinv = lax.rsqrt(deg)
    pos = _fast_sin(xyz_ref[...] * scales_ref[...])
    feat = jnp.concatenate(
        [pos, rot_ref[...], col_ref[...], sc_ref[...], op_ref[...]], axis=1)
    z = (jnp.dot(x_ref[...].astype(jnp.bfloat16), wx_ref[...],
                 preferred_element_type=jnp.float32)
         + jnp.dot(feat.astype(jnp.bfloat16), wf_ref[...],
                   preferred_element_type=jnp.float32))
    zp_ref[...] = z * dinv
    dinv_ref[...] = dinv


def _lin1(x, xyz, rot, color, scale, opacity, w1x, w1f, scales, degp1):
    row = lambda i: (i, 0)
    return pl.pallas_call(
        _lin1_body,
        grid=(N // BR,),
        in_specs=[
            pl.BlockSpec((BR, 288), row),
            pl.BlockSpec((BR, 3), row),
            pl.BlockSpec((BR, 4), row),
            pl.BlockSpec((BR, 3), row),
            pl.BlockSpec((BR, 3), row),
            pl.BlockSpec((BR, 1), row),
            pl.BlockSpec((288, H), lambda i: (0, 0)),
            pl.BlockSpec((14, H), lambda i: (0, 0)),
            pl.BlockSpec((1, 3), lambda i: (0, 0)),
            pl.BlockSpec((BR, 1), row),
        ],
        out_specs=[
            pl.BlockSpec((BR, H), row),
            pl.BlockSpec((BR, 1), row),
        ],
        out_shape=[
            jax.ShapeDtypeStruct((NPAD, H), jnp.float32),
            jax.ShapeDtypeStruct((NPAD, 1), jnp.float32),
        ],
    )(x, xyz, rot, color, scale, opacity, w1x, w1f, scales, degp1)


def _mid_body(acc_ref, dinv_ref, b_ref, w_ref, out_ref):
    dinv = dinv_ref[...]
    hcur = jnp.maximum(
        dinv * (acc_ref[0] + acc_ref[1]) + b_ref[...], 0.0)
    z = jnp.dot(hcur, w_ref[...], preferred_element_type=jnp.float32)
    out_ref[...] = z * dinv


def _mid(acc, dinv, b, w):
    return pl.pallas_call(
        _mid_body,
        grid=(N // BR,),
        in_specs=[
            pl.BlockSpec((2, BR, H), lambda i: (0, i, 0)),
            pl.BlockSpec((BR, 1), lambda i: (i, 0)),
            pl.BlockSpec((1, H), lambda i: (0, 0)),
            pl.BlockSpec((H, H), lambda i: (0, 0)),
        ],
        out_specs=pl.BlockSpec((BR, H), lambda i: (i, 0)),
        out_shape=jax.ShapeDtypeStruct((NPAD, H), jnp.float32),
    )(acc, dinv, b, w)


def _heads_body(acc_ref, dinv_ref, b3_ref, w1c_ref, b1c_ref,
                w2b_ref, b2c_ref, st_ref, dscale_ref, lo_ref, hi_ref,
                xyz_o, rot_o, col_o, sc_o, op_o):
    dinv = dinv_ref[...]
    h3 = jnp.maximum(
        dinv * (acc_ref[0] + acc_ref[1]) + b3_ref[...], 0.0)
    hh = jnp.maximum(
        jnp.dot(h3.astype(jnp.bfloat16), w1c_ref[...],
                preferred_element_type=jnp.float32)
        + b1c_ref[...], 0.0)
    delta = (jnp.dot(hh.astype(jnp.bfloat16), w2b_ref[...],
                     preferred_element_type=jnp.float32)
             + b2c_ref[...])
    new = st_ref[...] + delta * dscale_ref[...]
    out = jnp.clip(new, lo_ref[...], hi_ref[...])
    rn = new[:, 3:7]
    msk = (lax.broadcasted_iota(jnp.int32, new.shape, 1) - 3).astype(jnp.uint32) < 4
    nrm = jnp.sqrt(jnp.sum(jnp.where(msk, new * new, 0.0), axis=1,
                           keepdims=True))
    xyz_o[...] = out[:, 0:3]
    rot_o[...] = rn / jnp.clip(nrm, 1e-12, None)
    col_o[...] = out[:, 7:10]
    sc_o[...] = out[:, 10:13]
    op_o[...] = out[:, 13:14]


def _heads(acc, dinv, b3, w1c, b1c, w2b, b2c, state, dscale, lo, hi):
    row = lambda i: (i, 0)
    full = lambda shp: pl.BlockSpec(shp, lambda i: (0, 0))
    return pl.pallas_call(
        _heads_body,
        grid=(N // BR,),
        in_specs=[
            pl.BlockSpec((2, BR, H), lambda i: (0, i, 0)),
            pl.BlockSpec((BR, 1), row),
            full((1, H)),
            full((H, 320)),
            full((1, 320)),
            full((320, 14)),
            full((1, 14)),
            pl.BlockSpec((BR, 14), row),
            full((1, 14)),
            full((1, 14)),
            full((1, 14)),
        ],
        out_specs=[
            pl.BlockSpec((BR, 3), row),
            pl.BlockSpec((BR, 4), row),
            pl.BlockSpec((BR, 3), row),
            pl.BlockSpec((BR, 3), row),
            pl.BlockSpec((BR, 1), row),
        ],
        out_shape=[
            jax.ShapeDtypeStruct((N, 3), jnp.float32),
            jax.ShapeDtypeStruct((N, 4), jnp.float32),
            jax.ShapeDtypeStruct((N, 3), jnp.float32),
            jax.ShapeDtypeStruct((N, 3), jnp.float32),
            jax.ShapeDtypeStruct((N, 1), jnp.float32),
        ],
    )(acc, dinv, b3, w1c, b1c, w2b, b2c, state, dscale, lo, hi)


def kernel(x, xyz, rot, color, scale, opacity, edge_index,
           W1, b1, W2, b2, W3, b3,
           Wp1, bp1, Wp2, bp2, Wr1, br1, Wr2, br2,
           Wc1, bc1, Wc2, bc2, Ws1, bs1, Ws2, bs2,
           Wo1, bo1, Wo2, bo2):
    src_r = edge_index[0].reshape(NW, NB, K)
    dst_r = edge_index[1].reshape(NW, NB, K)
    dst_d = edge_index[1].reshape(NS, NBD, K)
    ones_k = jnp.ones((K,), jnp.float32)
    zeros_d = jnp.zeros((DPT,), jnp.float32)
    zeros_rows = jnp.zeros((RPT, H), jnp.float32)
    w1x = W1[:288].astype(jnp.bfloat16)
    w1f = W1[288:].astype(jnp.bfloat16)

    w1c = jnp.concatenate(
        [Wp1, Wr1, Wc1, Ws1, Wo1], axis=1).astype(jnp.bfloat16)
    b1c = jnp.concatenate([bp1, br1, bc1, bs1, bo1])[None, :]
    w2b = jnp.zeros((320, 14), jnp.float32)
    w2b = w2b.at[0:64, 0:3].set(Wp2)
    w2b = w2b.at[64:128, 3:7].set(Wr2)
    w2b = w2b.at[128:192, 7:10].set(Wc2)
    w2b = w2b.at[192:256, 10:13].set(Ws2)
    w2b = w2b.at[256:320, 13:14].set(Wo2)
    w2b = w2b.astype(jnp.bfloat16)
    b2c = jnp.concatenate([bp2, br2, bc2, bs2, bo2])[None, :]

    degp1 = _deg_kernel(dst_d, ones_k, zeros_d)[:, None]

    scales = jnp.array([[10.0, 20.0, 30.0]], jnp.float32)
    z1p, dinv = _lin1(x, xyz, rot, color, scale, opacity, w1x, w1f,
                      scales, degp1)
    acc1 = _agg_kernel(z1p, src_r, dst_r, zeros_rows).reshape(NC, NPAD, H)
    z2p = _mid(acc1, dinv, b1[None, :], W2)
    acc2 = _agg_kernel(z2p, src_r, dst_r, zeros_rows).reshape(NC, NPAD, H)
    z3p = _mid(acc2, dinv, b2[None, :], W3)
    acc3 = _agg_kernel(z3p, src_r, dst_r, zeros_rows).reshape(NC, NPAD, H)
    state = jnp.concatenate([xyz, rot, color, scale, opacity], axis=1)
    inf = jnp.inf
    dscale = jnp.array([[0.1] * 3 + [1.0] * 4 + [1.0] * 3 + [0.1] * 3 + [1.0]],
                       jnp.float32)
    lo = jnp.array([[-inf] * 7 + [0.0] * 3 + [1e-6] * 3 + [0.0]], jnp.float32)
    hi = jnp.array([[inf] * 7 + [1.0] * 3 + [1000.0] * 3 + [1.0]], jnp.float32)
    return _heads(acc3, dinv, b3[None, :], w1c, b1c, w2b, b2c,
                  state, dscale, lo, hi)


# speedup vs baseline: 1.0963x; 1.0216x over previous
import functools

import jax
import jax.numpy as jnp
from jax import lax
from jax.experimental import pallas as pl
from jax.experimental.pallas import tpu as pltpu
from jax.experimental.pallas import tpu_sc as plsc

N = 10000
E = 160000
H = 128

NC = 2
NS = 16
NW = NC * NS
EPW = E // NW
K = 125
NB = EPW // K
NPAD = 10240
RPT = NPAD // NS
DPT = NPAD // NS


NBD = E // NS // K


def _deg_body(dst_hbm, ones_hbm, zeros_hbm, out_hbm, idx_v, ones_v, acc_sh, sem):
    c = lax.axis_index("c")
    s = lax.axis_index("s")

    @pl.when(c == 0)
    def _():
        pltpu.sync_copy(dst_hbm.at[s], idx_v)
        pltpu.sync_copy(ones_hbm, ones_v)
        pltpu.sync_copy(zeros_hbm, acc_sh.at[pl.ds(s * DPT, DPT)])
        plsc.subcore_barrier()

        def body(j, carry):
            pltpu.sync_copy(ones_v, acc_sh.at[idx_v.at[j]], add=True)
            return carry

        lax.fori_loop(0, NBD, body, 0)
        plsc.subcore_barrier()
        pltpu.sync_copy(acc_sh.at[pl.ds(s * DPT, DPT)],
                        out_hbm.at[pl.ds(s * DPT, DPT)])


_deg_kernel = functools.partial(
    pl.kernel,
    out_type=jax.ShapeDtypeStruct((NPAD,), jnp.float32),
    mesh=plsc.VectorSubcoreMesh(core_axis_name="c", subcore_axis_name="s", num_cores=NC, num_subcores=NS),
    scratch_types=[
        pltpu.VMEM((NBD, K), jnp.int32),
        pltpu.VMEM((K,), jnp.float32),
        pltpu.VMEM_SHARED((NPAD,), jnp.float32),
        pltpu.SemaphoreType.DMA,
    ],
)(_deg_body)


def _agg_body(zp_hbm, src_hbm, dst_hbm, zrows_hbm, out_hbm,
              sidx, didx, rows0, rows1, acc_sh, gsem0, gsem1):
    c = lax.axis_index("c")
    s = lax.axis_index("s")
    w = c * NS + s
    pltpu.sync_copy(src_hbm.at[w], sidx)
    pltpu.sync_copy(dst_hbm.at[w], didx)

    @pl.when(c == 0)
    def _():
        pltpu.sync_copy(zp_hbm.at[pl.ds(s * RPT, RPT)],
                        acc_sh.at[pl.ds(s * RPT, RPT)])

    @pl.when(c == 1)
    def _():
        pltpu.sync_copy(zrows_hbm, acc_sh.at[pl.ds(s * RPT, RPT)])

    plsc.subcore_barrier()

    pltpu.async_copy(zp_hbm.at[sidx.at[0]], rows0, gsem0)

    def body(t, carry):
        j0 = 2 * t
        pltpu.make_async_copy(zp_hbm.at[sidx.at[j0]], rows0, gsem0).wait()
        pltpu.async_copy(zp_hbm.at[sidx.at[j0 + 1]], rows1, gsem1)
        pltpu.sync_copy(rows0, acc_sh.at[didx.at[j0]], add=True)
        pltpu.make_async_copy(zp_hbm.at[sidx.at[j0 + 1]], rows1, gsem1).wait()

        @pl.when(t < NB // 2 - 1)
        def _():
            pltpu.async_copy(zp_hbm.at[sidx.at[j0 + 2]], rows0, gsem0)

        pltpu.sync_copy(rows1, acc_sh.at[didx.at[j0 + 1]], add=True)
        return carry

    lax.fori_loop(0, NB // 2, body, 0)
    plsc.subcore_barrier()
    base = c * NPAD + s * RPT
    pltpu.sync_copy(acc_sh.at[pl.ds(s * RPT, RPT)], out_hbm.at[pl.ds(base, RPT)])


_agg_kernel = functools.partial(
    pl.kernel,
    out_type=jax.ShapeDtypeStruct((NC * NPAD, H), jnp.float32),
    mesh=plsc.VectorSubcoreMesh(core_axis_name="c", subcore_axis_name="s", num_cores=NC, num_subcores=NS),
    scratch_types=[
        pltpu.VMEM((NB, K), jnp.int32),
        pltpu.VMEM((NB, K), jnp.int32),
        pltpu.VMEM((K, H), jnp.float32),
        pltpu.VMEM((K, H), jnp.float32),
        pltpu.VMEM_SHARED((NPAD, H), jnp.float32),
        pltpu.SemaphoreType.DMA,
        pltpu.SemaphoreType.DMA,
    ],
)(_agg_body)


BR = 2000


def _fast_sin(x):
    k = jnp.round(x * (1.0 / jnp.pi))
    r = x - k * jnp.pi
    ki = k.astype(jnp.int32)
    sgn = jnp.where((ki & 1) == 0, 1.0, -1.0)
    r2 = r * r
    p = r * (1.0 + r2 * (-0.16666667 + r2 * (8.3333310e-3
                                             + r2 * (-1.9840874e-4
                                                     + r2 * 2.7525562e-6))))
    return sgn * p


def _lin1_body(x_ref, xyz_ref, rot_ref, col_ref, sc_ref, op_ref,
               wx_ref, wf_ref, scales_ref, degp_ref, zp_ref, dinv_ref):
    deg = degp_ref[...] + 1.0
    dinv = lax.rsqrt(deg)
    pos = _fast_sin(xyz_ref[...] * scales_ref[...])
    feat = jnp.concatenate(
        [pos, rot_ref[...], col_ref[...], sc_ref[...], op_ref[...]], axis=1)
    z = (jnp.dot(x_ref[...].astype(jnp.bfloat16), wx_ref[...],
                 preferred_element_type=jnp.float32)
         + jnp.dot(feat.astype(jnp.bfloat16), wf_ref[...],
                   preferred_element_type=jnp.float32))
    zp_ref[...] = z * dinv
    dinv_ref[...] = dinv


def _lin1(x, xyz, rot, color, scale, opacity, w1x, w1f, scales, degp1):
    row = lambda i: (i, 0)
    return pl.pallas_call(
        _lin1_body,
        grid=(N // BR,),
        in_specs=[
            pl.BlockSpec((BR, 288), row),
            pl.BlockSpec((BR, 3), row),
            pl.BlockSpec((BR, 4), row),
            pl.BlockSpec((BR, 3), row),
            pl.BlockSpec((BR, 3), row),
            pl.BlockSpec((BR, 1), row),
            pl.BlockSpec((288, H), lambda i: (0, 0)),
            pl.BlockSpec((14, H), lambda i: (0, 0)),
            pl.BlockSpec((1, 3), lambda i: (0, 0)),
            pl.BlockSpec((BR, 1), row),
        ],
        out_specs=[
            pl.BlockSpec((BR, H), row),
            pl.BlockSpec((BR, 1), row),
        ],
        out_shape=[
            jax.ShapeDtypeStruct((NPAD, H), jnp.float32),
            jax.ShapeDtypeStruct((NPAD, 1), jnp.float32),
        ],
    )(x, xyz, rot, color, scale, opacity, w1x, w1f, scales, degp1)


def _mid_body(acc_ref, dinv_ref, b_ref, w_ref, out_ref):
    dinv = dinv_ref[...]
    hcur = jnp.maximum(
        dinv * (acc_ref[0] + acc_ref[1]) + b_ref[...], 0.0)
    z = jnp.dot(hcur, w_ref[...], preferred_element_type=jnp.float32)
    out_ref[...] = z * dinv


def _mid(acc, dinv, b, w):
    return pl.pallas_call(
        _mid_body,
        grid=(N // BR,),
        in_specs=[
            pl.BlockSpec((2, BR, H), lambda i: (0, i, 0)),
            pl.BlockSpec((BR, 1), lambda i: (i, 0)),
            pl.BlockSpec((1, H), lambda i: (0, 0)),
            pl.BlockSpec((H, H), lambda i: (0, 0)),
        ],
        out_specs=pl.BlockSpec((BR, H), lambda i: (i, 0)),
        out_shape=jax.ShapeDtypeStruct((NPAD, H), jnp.float32),
    )(acc, dinv, b, w)


def _heads_body(acc_ref, dinv_ref, b3_ref, w1c_ref, b1c_ref,
                w2b_ref, b2c_ref, st_ref, dscale_ref, lo_ref, hi_ref,
                xyz_o, rot_o, col_o, sc_o, op_o):
    dinv = dinv_ref[...]
    h3 = jnp.maximum(
        dinv * (acc_ref[0] + acc_ref[1]) + b3_ref[...], 0.0)
    hh = jnp.maximum(
        jnp.dot(h3.astype(jnp.bfloat16), w1c_ref[...],
                preferred_element_type=jnp.float32)
        + b1c_ref[...], 0.0)
    delta = (jnp.dot(hh.astype(jnp.bfloat16), w2b_ref[...],
                     preferred_element_type=jnp.float32)
             + b2c_ref[...])
    new = st_ref[...] + delta * dscale_ref[...]
    out = jnp.clip(new, lo_ref[...], hi_ref[...])
    rn = new[:, 3:7]
    msk = (lax.broadcasted_iota(jnp.int32, new.shape, 1) - 3).astype(jnp.uint32) < 4
    nrm = jnp.sqrt(jnp.sum(jnp.where(msk, new * new, 0.0), axis=1,
                           keepdims=True))
    xyz_o[...] = out[:, 0:3]
    rot_o[...] = rn / jnp.clip(nrm, 1e-12, None)
    col_o[...] = out[:, 7:10]
    sc_o[...] = out[:, 10:13]
    op_o[...] = out[:, 13:14]


def _heads(acc, dinv, b3, w1c, b1c, w2b, b2c, state, dscale, lo, hi):
    row = lambda i: (i, 0)
    full = lambda shp: pl.BlockSpec(shp, lambda i: (0, 0))
    return pl.pallas_call(
        _heads_body,
        grid=(N // BR,),
        in_specs=[
            pl.BlockSpec((2, BR, H), lambda i: (0, i, 0)),
            pl.BlockSpec((BR, 1), row),
            full((1, H)),
            full((H, 320)),
            full((1, 320)),
            full((320, 14)),
            full((1, 14)),
            pl.BlockSpec((BR, 14), row),
            full((1, 14)),
            full((1, 14)),
            full((1, 14)),
        ],
        out_specs=[
            pl.BlockSpec((BR, 3), row),
            pl.BlockSpec((BR, 4), row),
            pl.BlockSpec((BR, 3), row),
            pl.BlockSpec((BR, 3), row),
            pl.BlockSpec((BR, 1), row),
        ],
        out_shape=[
            jax.ShapeDtypeStruct((N, 3), jnp.float32),
            jax.ShapeDtypeStruct((N, 4), jnp.float32),
            jax.ShapeDtypeStruct((N, 3), jnp.float32),
            jax.ShapeDtypeStruct((N, 3), jnp.float32),
            jax.ShapeDtypeStruct((N, 1), jnp.float32),
        ],
    )(acc, dinv, b3, w1c, b1c, w2b, b2c, state, dscale, lo, hi)


def kernel(x, xyz, rot, color, scale, opacity, edge_index,
           W1, b1, W2, b2, W3, b3,
           Wp1, bp1, Wp2, bp2, Wr1, br1, Wr2, br2,
           Wc1, bc1, Wc2, bc2, Ws1, bs1, Ws2, bs2,
           Wo1, bo1, Wo2, bo2):
    src_r = edge_index[0].reshape(NW, NB, K)
    dst_r = edge_index[1].reshape(NW, NB, K)
    dst_d = edge_index[1].reshape(NS, NBD, K)
    ones_k = jnp.ones((K,), jnp.float32)
    zeros_d = jnp.zeros((DPT,), jnp.float32)
    zeros_rows = jnp.zeros((RPT, H), jnp.float32)
    w1x = W1[:288].astype(jnp.bfloat16)
    w1f = W1[288:].astype(jnp.bfloat16)

    w1c = jnp.concatenate(
        [Wp1, Wr1, Wc1, Ws1, Wo1], axis=1).astype(jnp.bfloat16)
    b1c = jnp.concatenate([bp1, br1, bc1, bs1, bo1])[None, :]
    w2b = jnp.zeros((320, 14), jnp.float32)
    w2b = w2b.at[0:64, 0:3].set(Wp2)
    w2b = w2b.at[64:128, 3:7].set(Wr2)
    w2b = w2b.at[128:192, 7:10].set(Wc2)
    w2b = w2b.at[192:256, 10:13].set(Ws2)
    w2b = w2b.at[256:320, 13:14].set(Wo2)
    w2b = w2b.astype(jnp.bfloat16)
    b2c = jnp.concatenate([bp2, br2, bc2, bs2, bo2])[None, :]

    degp1 = _deg_kernel(dst_d, ones_k, zeros_d)[:, None]

    scales = jnp.array([[10.0, 20.0, 30.0]], jnp.float32)
    z1p, dinv = _lin1(x, xyz, rot, color, scale, opacity, w1x, w1f,
                      scales, degp1)
    acc1 = _agg_kernel(z1p, src_r, dst_r, zeros_rows).reshape(NC, NPAD, H)
    z2p = _mid(acc1, dinv, b1[None, :], W2)
    acc2 = _agg_kernel(z2p, src_r, dst_r, zeros_rows).reshape(NC, NPAD, H)
    z3p = _mid(acc2, dinv, b2[None, :], W3)
    acc3 = _agg_kernel(z3p, src_r, dst_r, zeros_rows).reshape(NC, NPAD, H)
    state = jnp.concatenate([xyz, rot, color, scale, opacity], axis=1)
    inf = jnp.inf
    dscale = jnp.array([[0.1] * 3 + [1.0] * 4 + [1.0] * 3 + [0.1] * 3 + [1.0]],
                       jnp.float32)
    lo = jnp.array([[-inf] * 7 + [0.0] * 3 + [1e-6] * 3 + [0.0]], jnp.float32)
    hi = jnp.array([[inf] * 7 + [1.0] * 3 + [1000.0] * 3 + [1.0]], jnp.float32)
    return _heads(acc3, dinv, b3[None, :], w1c, b1c, w2b, b2c,
                  state, dscale, lo, hi)
